# Initial kernel scaffold; baseline (speedup 1.0000x reference)
import jax
import jax.numpy as jnp
from jax.experimental import pallas as pl


def kernel(cam, true_mask):
    raise NotImplementedError("write your pallas kernel here")


# speedup vs baseline: 5.1249x; 5.1249x over previous
import numpy as np
import jax
import jax.numpy as jnp
from jax.experimental import pallas as pl
from jax.experimental.pallas import tpu as pltpu

_HW = 512 * 512
_K = 2048
_NZ = 200
_ZS_NP = np.arange(10, 2001, 10).astype(np.int32)
_BLK = 128
_NBLK = _K // _BLK


def _acc_seeds_kernel(tv_r, tv_c, ti_r, ti_c, tf_c,
                      bv_r, bv_c, bi_r, bi_c, bf_c,
                      zs_ref, af_ref, ab_ref, t_top, t_bot):
    zs = zs_ref[...]

    def fill_ranks(v_r_ref, v_c_ref, i_r_ref, i_c_ref, t_ref, top_side):
        v_row = v_r_ref[...]
        i_row = i_r_ref[...]

        def body(b, carry):
            vb = v_c_ref[pl.ds(b * _BLK, _BLK), :]
            ib = i_c_ref[pl.ds(b * _BLK, _BLK), :]
            if top_side:
                beats = (v_row > vb) | ((v_row == vb) & (i_row > ib))
            else:
                beats = (v_row < vb) | ((v_row == vb) & (i_row < ib))
            tb = jnp.sum(beats.astype(jnp.float32), axis=1, keepdims=True)
            t_ref[pl.ds(b * _BLK, _BLK), :] = tb
            return carry

        jax.lax.fori_loop(0, _NBLK, body, 0)

    fill_ranks(tv_r, tv_c, ti_r, ti_c, t_top, True)
    fill_ranks(bv_r, bv_c, bi_r, bi_c, t_bot, False)

    tt = t_top[...]
    tb = t_bot[...]
    mem_t = (tt < zs).astype(jnp.float32)
    mem_b = (tb < zs).astype(jnp.float32)
    s_top = jnp.sum(mem_t * tf_c[...], axis=0, keepdims=True)
    s_bot = jnp.sum(mem_b * bf_c[...], axis=0, keepdims=True)
    af_ref[...] = 100.0 * s_top / zs
    ab_ref[...] = 100.0 * s_bot / zs


@jax.jit
def kernel(cam, true_mask):
    cam_flat = cam.reshape(_HW)
    forg = true_mask.reshape(_HW)
    backg = 1.0 - forg

    tv, ti = jax.lax.top_k(cam_flat, _K)
    nbv, bi = jax.lax.top_k(-cam_flat, _K)
    bv = -nbv
    tf = forg[ti]
    bf = backg[bi]

    zs = jnp.asarray(_ZS_NP, dtype=jnp.float32).reshape(1, _NZ)

    af, ab = pl.pallas_call(
        _acc_seeds_kernel,
        out_shape=[jax.ShapeDtypeStruct((1, _NZ), jnp.float32),
                   jax.ShapeDtypeStruct((1, _NZ), jnp.float32)],
        scratch_shapes=[pltpu.VMEM((_K, 1), jnp.float32),
                        pltpu.VMEM((_K, 1), jnp.float32)],
    )(tv.reshape(1, _K), tv.reshape(_K, 1),
      ti.reshape(1, _K), ti.reshape(_K, 1), tf.reshape(_K, 1),
      bv.reshape(1, _K), bv.reshape(_K, 1),
      bi.reshape(1, _K), bi.reshape(_K, 1), bf.reshape(_K, 1),
      zs)

    return (af.reshape(_NZ), ab.reshape(_NZ), jnp.asarray(_ZS_NP))
